# Initial kernel scaffold; baseline (speedup 1.0000x reference)
import jax
import jax.numpy as jnp
from jax.experimental import pallas as pl


def kernel(x, token_table, pos_table):
    raise NotImplementedError("write your pallas kernel here")


# speedup vs baseline: 3.5678x; 3.5678x over previous
import functools

import jax
import jax.numpy as jnp
from jax import lax
from jax.experimental import pallas as pl
from jax.experimental.pallas import tpu as pltpu
from jax.experimental.pallas import tpu_sc as plsc

_NC = 2
_NS = 16
_NW = _NC * _NS


@functools.lru_cache(maxsize=None)
def _build(B, S, D):
    ch_seq = 2
    CH = ch_seq * S
    tok_per_w = (B // _NW) * S
    n_ch = tok_per_w // CH

    mesh = plsc.VectorSubcoreMesh(core_axis_name="c", subcore_axis_name="s")

    def body(xf_hbm, table_hbm, pos2_hbm, out_hbm, pos_s, idx_v, tok_v, gsem):
        sid = lax.axis_index("s")
        wid = sid * _NC + lax.axis_index("c")
        base = wid * tok_per_w

        @pl.when(sid == 0)
        def _():
            pltpu.sync_copy(pos2_hbm, pos_s)

        plsc.subcore_barrier()

        def chunk(g, carry):
            off = base + g * CH
            pltpu.sync_copy(xf_hbm.at[pl.ds(off, CH)], idx_v)
            pltpu.sync_copy(pos_s, tok_v)
            pltpu.async_copy(table_hbm.at[idx_v], tok_v, gsem, add=True).wait()
            pltpu.sync_copy(tok_v, out_hbm.at[pl.ds(off, CH)])
            return carry

        lax.fori_loop(0, n_ch, chunk, 0)

    return pl.kernel(
        body,
        out_type=jax.ShapeDtypeStruct((B * S, D), jnp.float32),
        mesh=mesh,
        compiler_params=pltpu.CompilerParams(use_tc_tiling_on_sc=False),
        scratch_types=[
            pltpu.VMEM_SHARED((CH, D), jnp.float32),
            pltpu.VMEM((CH,), jnp.int32),
            pltpu.VMEM((CH, D), jnp.float32),
            pltpu.SemaphoreType.DMA,
        ],
    )


def kernel(x, token_table, pos_table):
    B, S = x.shape
    V, D = token_table.shape
    xf = x.reshape(B * S).astype(jnp.int32)
    pos2 = jnp.concatenate([pos_table, pos_table], axis=0)
    out2 = _build(B, S, D)(xf, token_table, pos2)
    return out2.reshape(B, S, D)


# speedup vs baseline: 4.2517x; 1.1917x over previous
import functools

import jax
import jax.numpy as jnp
from jax import lax
from jax.experimental import pallas as pl
from jax.experimental.pallas import tpu as pltpu
from jax.experimental.pallas import tpu_sc as plsc

_NC = 2
_NS = 16
_NW = _NC * _NS
_NB = 4


@functools.lru_cache(maxsize=None)
def _build(B, S, D):
    ch_seq = 2
    CH = ch_seq * S
    tok_per_w = (B // _NW) * S
    n_ch = tok_per_w // CH

    mesh = plsc.VectorSubcoreMesh(core_axis_name="c", subcore_axis_name="s")

    def body(xf_hbm, table_hbm, pos2_hbm, out_hbm,
             pos_s, idx_v, tok_v, isem, psem, gsem, osem):
        sid = lax.axis_index("s")
        wid = sid * _NC + lax.axis_index("c")
        base = wid * tok_per_w

        @pl.when(sid == 0)
        def _():
            pltpu.sync_copy(pos2_hbm, pos_s)

        plsc.subcore_barrier()

        def slot(s, carry):
            @pl.when(s < n_ch)
            def _():
                bP = lax.rem(s, _NB)

                @pl.when(s >= _NB)
                def _():
                    pltpu.make_async_copy(
                        tok_v.at[bP], out_hbm.at[pl.ds(0, CH)], osem.at[bP]
                    ).wait()

                off = base + s * CH
                pltpu.async_copy(pos_s, tok_v.at[bP], psem.at[bP])
                pltpu.async_copy(xf_hbm.at[pl.ds(off, CH)], idx_v.at[bP],
                                 isem.at[bP])

            @pl.when(jnp.logical_and(s >= 1, s <= n_ch))
            def _():
                bG = lax.rem(s - 1, _NB)
                pltpu.make_async_copy(pos_s, tok_v.at[bG], psem.at[bG]).wait()
                pltpu.make_async_copy(xf_hbm.at[pl.ds(0, CH)], idx_v.at[bG],
                                      isem.at[bG]).wait()
                pltpu.async_copy(table_hbm.at[idx_v.at[bG]], tok_v.at[bG],
                                 gsem.at[bG], add=True)

            @pl.when(s >= 2)
            def _():
                o = s - 2
                bO = lax.rem(o, _NB)
                pltpu.make_async_copy(table_hbm.at[idx_v.at[bO]], tok_v.at[bO],
                                      gsem.at[bO]).wait()
                off = base + o * CH
                pltpu.async_copy(tok_v.at[bO], out_hbm.at[pl.ds(off, CH)],
                                 osem.at[bO])

            return carry

        lax.fori_loop(0, n_ch + 2, slot, 0)

        for b in range(_NB):
            pltpu.make_async_copy(
                tok_v.at[b], out_hbm.at[pl.ds(0, CH)], osem.at[b]
            ).wait()

    return pl.kernel(
        body,
        out_type=jax.ShapeDtypeStruct((B * S, D), jnp.float32),
        mesh=mesh,
        compiler_params=pltpu.CompilerParams(use_tc_tiling_on_sc=False),
        scratch_types=[
            pltpu.VMEM_SHARED((CH, D), jnp.float32),
            pltpu.VMEM((_NB, CH), jnp.int32),
            pltpu.VMEM((_NB, CH, D), jnp.float32),
            pltpu.SemaphoreType.DMA((_NB,)),
            pltpu.SemaphoreType.DMA((_NB,)),
            pltpu.SemaphoreType.DMA((_NB,)),
            pltpu.SemaphoreType.DMA((_NB,)),
        ],
    )


def kernel(x, token_table, pos_table):
    B, S = x.shape
    V, D = token_table.shape
    xf = x.reshape(B * S).astype(jnp.int32)
    pos2 = jnp.concatenate([pos_table, pos_table], axis=0)
    out2 = _build(B, S, D)(xf, token_table, pos2)
    return out2.reshape(B, S, D)
